# baseline (speedup 1.0000x reference)
import functools

import jax
import jax.numpy as jnp
import numpy as np
from jax import lax
from jax.experimental import pallas as pl
from jax.experimental.pallas import tpu as pltpu
from jax.experimental.pallas import tpu_sc as plsc

VOCAB = 1000000
D = 64
B = 4096
L = 200
BL = B * L

NC = 2
NS = 16
NW = NC * NS
ROWS_PER_W = BL // NW
N_CH = ROWS_PER_W // L


def _positional_encoding_np(seq_len, d_model):
    pos = np.arange(seq_len, dtype=np.float32)[:, None]
    i = np.arange(0, d_model, 2, dtype=np.float32)[None, :]
    angles = pos / np.power(10000.0, i / d_model)
    pe = np.zeros((seq_len, d_model), dtype=np.float32)
    pe[:, 0::2] = np.sin(angles)
    pe[:, 1::2] = np.cos(angles)
    return pe


_MESH = plsc.VectorSubcoreMesh(
    core_axis_name="c", subcore_axis_name="s", num_cores=NC, num_subcores=NS
)


@functools.partial(
    pl.kernel,
    mesh=_MESH,
    out_type=jax.ShapeDtypeStruct((BL, D), jnp.float32),
    scratch_types=[
        pltpu.VMEM((2, 2, 128), jnp.int32),
        pltpu.VMEM((2, L, D), jnp.float32),
        pltpu.VMEM((L, D), jnp.float32),
        pltpu.SemaphoreType.DMA,
    ],
    compiler_params=pltpu.CompilerParams(use_tc_tiling_on_sc=False),
)
def _seq_embed(x_hbm, pe_hbm, table_hbm, out_hbm, idx_v, rows_v, pe_v, sem):
    wid = lax.axis_index("s") * NC + lax.axis_index("c")
    base = wid * ROWS_PER_W
    pltpu.sync_copy(pe_hbm, pe_v)

    def step(g, carry):
        r0 = base + g * L
        pltpu.sync_copy(x_hbm.at[pl.ds(r0, 128)], idx_v.at[0, 0])
        pltpu.sync_copy(x_hbm.at[pl.ds(r0 + 128, 72)], idx_v.at[0, 1, pl.ds(0, 72)])
        c1 = pltpu.async_copy(
            table_hbm.at[idx_v.at[0, 0]], rows_v.at[0, pl.ds(0, 128)], sem
        )
        c2 = pltpu.async_copy(
            table_hbm.at[idx_v.at[0, 1, pl.ds(0, 72)]],
            rows_v.at[0, pl.ds(128, 72)],
            sem,
        )
        c1.wait()
        c2.wait()

        def add_row(r, c2_):
            for c in range(D // 16):
                s = pl.ds(c * 16, 16)
                rows_v[0, r, s] = rows_v[0, r, s] + pe_v[r, s]
            return c2_

        lax.fori_loop(0, L, add_row, 0)
        pltpu.sync_copy(rows_v.at[0], out_hbm.at[pl.ds(r0, L)])
        return carry

    lax.fori_loop(0, N_CH, step, 0)


def kernel(x, table):
    pe = jnp.asarray(_positional_encoding_np(L, D))
    x_flat = x.reshape(-1).astype(jnp.int32)
    out = _seq_embed(x_flat, pe, table)
    return out.reshape(B, L, D)
